# Initial kernel scaffold; baseline (speedup 1.0000x reference)
import jax
import jax.numpy as jnp
from jax.experimental import pallas as pl


def kernel(s1, s2, e1_weight, e2_weight):
    raise NotImplementedError("write your pallas kernel here")


# speedup vs baseline: 2.3231x; 2.3231x over previous
import functools

import jax
import jax.numpy as jnp
from jax import lax
from jax.experimental import pallas as pl
from jax.experimental.pallas import tpu as pltpu
from jax.experimental.pallas import tpu_sc as plsc


def _make_sc_kernel(N, D):
    info = plsc.get_sparse_core_info()
    NC, NS, LN = info.num_cores, info.num_subcores, info.num_lanes
    NW = NC * NS
    assert N % NW == 0
    per_w = N // NW
    CH = 512
    assert per_w % CH == 0
    n_ch = per_w // CH
    mesh = plsc.VectorSubcoreMesh(core_axis_name="c", subcore_axis_name="s")

    @functools.partial(
        pl.kernel,
        mesh=mesh,
        out_type=jax.ShapeDtypeStruct((N, D), jnp.float32),
        scratch_types=[
            pltpu.VMEM((CH,), jnp.int32),
            pltpu.VMEM((CH,), jnp.int32),
            pltpu.VMEM((CH, D), jnp.float32),
            pltpu.VMEM((CH, D), jnp.float32),
            pltpu.SemaphoreType.DMA,
        ],
        compiler_params=pltpu.CompilerParams(use_tc_tiling_on_sc=False),
    )
    def k(s1_hbm, s2_hbm, t1_hbm, t2_hbm, out_hbm, idx1_v, idx2_v, buf1, buf2, sem):
        wid = lax.axis_index("s") * NC + lax.axis_index("c")
        base = wid * per_w

        def chunk(c, carry):
            off = base + c * CH
            pltpu.sync_copy(s1_hbm.at[pl.ds(off, CH)], idx1_v)
            pltpu.sync_copy(s2_hbm.at[pl.ds(off, CH)], idx2_v)
            cp1 = pltpu.async_copy(t1_hbm.at[idx1_v], buf1, sem)
            cp2 = pltpu.async_copy(t2_hbm.at[idx2_v], buf2, sem)
            cp1.wait()
            cp2.wait()

            def add_row(i, c2):
                for q in range(D // LN):
                    sl = pl.ds(q * LN, LN)
                    buf1[i, sl] = buf1[i, sl] + buf2[i, sl]
                return c2

            lax.fori_loop(0, CH, add_row, 0)
            pltpu.sync_copy(buf1, out_hbm.at[pl.ds(off, CH)])
            return carry

        lax.fori_loop(0, n_ch, chunk, 0)

    return k


def kernel(s1, s2, e1_weight, e2_weight):
    B, L = s1.shape
    D = e1_weight.shape[1]
    N = B * L
    s1f = s1.reshape(N).astype(jnp.int32)
    s2f = s2.reshape(N).astype(jnp.int32)
    k = _make_sc_kernel(N, D)
    out = k(s1f, s2f, e1_weight, e2_weight)
    return out.reshape(B, L, D)


# speedup vs baseline: 2.4486x; 1.0540x over previous
import functools

import jax
import jax.numpy as jnp
from jax import lax
from jax.experimental import pallas as pl
from jax.experimental.pallas import tpu as pltpu
from jax.experimental.pallas import tpu_sc as plsc


def _make_sc_kernel(N, D):
    info = plsc.get_sparse_core_info()
    NC, NS, LN = info.num_cores, info.num_subcores, info.num_lanes
    NW = NC * NS
    assert N % NW == 0
    per_w = N // NW
    CH = 512
    assert per_w % CH == 0
    n_ch = per_w // CH
    mesh = plsc.VectorSubcoreMesh(core_axis_name="c", subcore_axis_name="s")

    @functools.partial(
        pl.kernel,
        mesh=mesh,
        out_type=jax.ShapeDtypeStruct((N, D), jnp.float32),
        scratch_types=[
            pltpu.VMEM((CH,), jnp.int32),
            pltpu.VMEM((CH,), jnp.int32),
            pltpu.VMEM((CH, D), jnp.float32),
            pltpu.VMEM((CH, D), jnp.float32),
            pltpu.SemaphoreType.DMA,
        ],
        compiler_params=pltpu.CompilerParams(use_tc_tiling_on_sc=False),
    )
    def k(s1_hbm, s2_hbm, t1_hbm, t2_hbm, out_hbm, idx1_v, idx2_v, buf1, buf2, sem):
        wid = lax.axis_index("s") * NC + lax.axis_index("c")
        base = wid * per_w

        def chunk(c, carry):
            off = base + c * CH
            pltpu.sync_copy(s1_hbm.at[pl.ds(off, CH)], idx1_v)
            pltpu.sync_copy(s2_hbm.at[pl.ds(off, CH)], idx2_v)
            pltpu.async_copy(t1_hbm.at[idx1_v], buf1, sem).wait()
            pltpu.async_copy(t2_hbm.at[idx2_v], buf1, sem, add=True).wait()
            pltpu.sync_copy(buf1, out_hbm.at[pl.ds(off, CH)])
            return carry

        lax.fori_loop(0, n_ch, chunk, 0)

    return k


def kernel(s1, s2, e1_weight, e2_weight):
    B, L = s1.shape
    D = e1_weight.shape[1]
    N = B * L
    s1f = s1.reshape(N).astype(jnp.int32)
    s2f = s2.reshape(N).astype(jnp.int32)
    k = _make_sc_kernel(N, D)
    out = k(s1f, s2f, e1_weight, e2_weight)
    return out.reshape(B, L, D)


# speedup vs baseline: 2.6352x; 1.0762x over previous
import functools

import jax
import jax.numpy as jnp
from jax import lax
from jax.experimental import pallas as pl
from jax.experimental.pallas import tpu as pltpu
from jax.experimental.pallas import tpu_sc as plsc

_NSLOT = 4


def _make_sc_kernel(N, D):
    info = plsc.get_sparse_core_info()
    NC, NS, LN = info.num_cores, info.num_subcores, info.num_lanes
    NW = NC * NS
    assert N % NW == 0
    per_w = N // NW
    CH = 200
    assert per_w % CH == 0 and CH % 8 == 0
    n_ch = per_w // CH
    assert n_ch % _NSLOT == 0 and n_ch // _NSLOT >= 3
    mesh = plsc.VectorSubcoreMesh(core_axis_name="c", subcore_axis_name="s")

    @functools.partial(
        pl.kernel,
        mesh=mesh,
        out_type=jax.ShapeDtypeStruct((N, D), jnp.float32),
        scratch_types=(
            [pltpu.VMEM((CH,), jnp.int32)] * (2 * _NSLOT)
            + [
                pltpu.VMEM((_NSLOT, CH, D), jnp.float32),
                pltpu.VMEM((_NSLOT, CH, D), jnp.float32),
                pltpu.SemaphoreType.DMA((_NSLOT,)),
                pltpu.SemaphoreType.DMA((_NSLOT,)),
            ]
        ),
        compiler_params=pltpu.CompilerParams(use_tc_tiling_on_sc=False),
    )
    def k(s1_hbm, s2_hbm, t1_hbm, t2_hbm, out_hbm, *scratch):
        idx1 = scratch[0:_NSLOT]
        idx2 = scratch[_NSLOT : 2 * _NSLOT]
        bufa, bufb, isem, ssem = scratch[2 * _NSLOT :]
        wid = lax.axis_index("s") * NC + lax.axis_index("c")
        base = wid * per_w

        def idx_issue(c, s):
            off = base + c * CH
            pltpu.async_copy(s1_hbm.at[pl.ds(off, CH)], idx1[s], isem.at[s])
            pltpu.async_copy(s2_hbm.at[pl.ds(off, CH)], idx2[s], isem.at[s])

        def idx_wait(s):
            pltpu.make_async_copy(s1_hbm.at[pl.ds(0, CH)], idx1[s], isem.at[s]).wait()
            pltpu.make_async_copy(s2_hbm.at[pl.ds(0, CH)], idx2[s], isem.at[s]).wait()

        def g_issue(s):
            pltpu.async_copy(t1_hbm.at[idx1[s]], bufa.at[s], ssem.at[s])
            pltpu.async_copy(t2_hbm.at[idx2[s]], bufb.at[s], ssem.at[s])

        def g_wait(s):
            pltpu.make_async_copy(t1_hbm.at[idx1[s]], bufa.at[s], ssem.at[s]).wait()
            pltpu.make_async_copy(t2_hbm.at[idx2[s]], bufb.at[s], ssem.at[s]).wait()

        def out_issue(c, s):
            off = base + c * CH
            pltpu.async_copy(bufa.at[s], out_hbm.at[pl.ds(off, CH)], ssem.at[s])

        def out_wait(s):
            pltpu.make_async_copy(
                bufa.at[s], out_hbm.at[pl.ds(base, CH)], ssem.at[s]
            ).wait()

        def add_chunk(s):
            @plsc.parallel_loop(0, CH // 4, unroll=2)
            def _(i):
                r0 = i * 4
                for rr in range(4):
                    for q in range(D // LN):
                        sl = pl.ds(q * LN, LN)
                        bufa[s, r0 + rr, sl] = bufa[s, r0 + rr, sl] + bufb[s, r0 + rr, sl]

        def emit(c, j, w_idx, w_out, i_g, i_idx):
            sp1 = (j + 1) % _NSLOT
            sp3 = (j + 3) % _NSLOT
            if w_idx:
                idx_wait(sp1)
            if w_out:
                out_wait(sp1)
            if i_g:
                g_issue(sp1)
            g_wait(j)
            add_chunk(j)
            out_issue(c, j)
            if i_idx:
                idx_issue(c + 3, sp3)

        idx_issue(0, 0)
        idx_issue(1, 1)
        idx_issue(2, 2)
        idx_wait(0)
        g_issue(0)

        emit(0, 0, True, False, True, True)
        emit(1, 1, True, False, True, True)
        emit(2, 2, True, False, True, True)
        emit(3, 3, True, True, True, True)

        def body(kk, carry):
            c0 = kk * _NSLOT
            for j in range(_NSLOT):
                emit(c0 + j, j, True, True, True, True)
            return carry

        lax.fori_loop(1, n_ch // _NSLOT - 1, body, 0)

        cL = n_ch - _NSLOT
        emit(cL + 0, 0, True, True, True, True)
        emit(cL + 1, 1, True, True, True, False)
        emit(cL + 2, 2, True, True, True, False)
        emit(cL + 3, 3, False, True, False, False)

        out_wait(1)
        out_wait(2)
        out_wait(3)

    return k


def kernel(s1, s2, e1_weight, e2_weight):
    B, L = s1.shape
    D = e1_weight.shape[1]
    N = B * L
    s1f = s1.reshape(N).astype(jnp.int32)
    s2f = s2.reshape(N).astype(jnp.int32)
    k = _make_sc_kernel(N, D)
    out = k(s1f, s2f, e1_weight, e2_weight)
    return out.reshape(B, L, D)


# speedup vs baseline: 2.6378x; 1.0010x over previous
import functools

import jax
import jax.numpy as jnp
from jax import lax
from jax.experimental import pallas as pl
from jax.experimental.pallas import tpu as pltpu
from jax.experimental.pallas import tpu_sc as plsc

_NSLOT = 4


def _make_sc_kernel(N, D):
    info = plsc.get_sparse_core_info()
    NC, NS, LN = info.num_cores, info.num_subcores, info.num_lanes
    NW = NC * NS
    assert N % NW == 0
    per_w = N // NW
    CH = 200
    assert per_w % CH == 0 and CH % 8 == 0
    n_ch = per_w // CH
    assert n_ch % _NSLOT == 0 and n_ch // _NSLOT >= 3
    mesh = plsc.VectorSubcoreMesh(core_axis_name="c", subcore_axis_name="s")

    @functools.partial(
        pl.kernel,
        mesh=mesh,
        out_type=jax.ShapeDtypeStruct((N, D), jnp.float32),
        scratch_types=(
            [pltpu.VMEM((CH,), jnp.int32)] * (2 * _NSLOT)
            + [
                pltpu.VMEM((_NSLOT, CH, D), jnp.float32),
                pltpu.VMEM((_NSLOT, CH, D), jnp.float32),
            ]
            + [pltpu.SemaphoreType.DMA] * (2 * _NSLOT)
        ),
        compiler_params=pltpu.CompilerParams(use_tc_tiling_on_sc=False),
    )
    def k(s1_hbm, s2_hbm, t1_hbm, t2_hbm, out_hbm, *scratch):
        idx1 = scratch[0:_NSLOT]
        idx2 = scratch[_NSLOT : 2 * _NSLOT]
        bufa, bufb = scratch[2 * _NSLOT : 2 * _NSLOT + 2]
        isem = scratch[2 * _NSLOT + 2 : 2 * _NSLOT + 2 + _NSLOT]
        ssem = scratch[2 * _NSLOT + 2 + _NSLOT :]
        wid = lax.axis_index("s") * NC + lax.axis_index("c")
        base = wid * per_w

        def idx_issue(c, s):
            off = base + c * CH
            pltpu.async_copy(s1_hbm.at[pl.ds(off, CH)], idx1[s], isem[s])
            pltpu.async_copy(s2_hbm.at[pl.ds(off, CH)], idx2[s], isem[s])

        def idx_wait(s):
            pltpu.make_async_copy(s1_hbm.at[pl.ds(0, CH)], idx1[s], isem[s]).wait()
            pltpu.make_async_copy(s2_hbm.at[pl.ds(0, CH)], idx2[s], isem[s]).wait()

        def g_issue(s):
            pltpu.async_copy(t1_hbm.at[idx1[s]], bufa.at[s], ssem[s])
            pltpu.async_copy(t2_hbm.at[idx2[s]], bufb.at[s], ssem[s])

        def g_wait(s):
            pltpu.make_async_copy(t1_hbm.at[idx1[s]], bufa.at[s], ssem[s]).wait()
            pltpu.make_async_copy(t2_hbm.at[idx2[s]], bufb.at[s], ssem[s]).wait()

        def out_issue(c, s):
            off = base + c * CH
            pltpu.async_copy(bufa.at[s], out_hbm.at[pl.ds(off, CH)], ssem[s])

        def out_wait(s):
            pltpu.make_async_copy(
                bufa.at[s], out_hbm.at[pl.ds(base, CH)], ssem[s]
            ).wait()

        def add_chunk(s):
            def body(i, carry):
                r0 = i * 4
                for rr in range(4):
                    for q in range(D // LN):
                        sl = pl.ds(q * LN, LN)
                        bufa[s, r0 + rr, sl] = bufa[s, r0 + rr, sl] + bufb[s, r0 + rr, sl]
                return carry

            lax.fori_loop(0, CH // 4, body, 0)

        def emit(c, j, w_idx, w_out, i_g, i_idx):
            sp1 = (j + 1) % _NSLOT
            sp3 = (j + 3) % _NSLOT
            if w_idx:
                idx_wait(sp1)
            if w_out:
                out_wait(sp1)
            if i_g:
                g_issue(sp1)
            g_wait(j)
            add_chunk(j)
            out_issue(c, j)
            if i_idx:
                idx_issue(c + 3, sp3)

        idx_issue(0, 0)
        idx_issue(1, 1)
        idx_issue(2, 2)
        idx_wait(0)
        g_issue(0)

        emit(0, 0, True, False, True, True)
        emit(1, 1, True, False, True, True)
        emit(2, 2, True, False, True, True)
        emit(3, 3, True, True, True, True)

        def body(kk, carry):
            c0 = kk * _NSLOT
            for j in range(_NSLOT):
                emit(c0 + j, j, True, True, True, True)
            return carry

        lax.fori_loop(1, n_ch // _NSLOT - 1, body, 0)

        cL = n_ch - _NSLOT
        emit(cL + 0, 0, True, True, True, True)
        emit(cL + 1, 1, True, True, True, False)
        emit(cL + 2, 2, True, True, True, False)
        emit(cL + 3, 3, False, True, False, False)

        out_wait(1)
        out_wait(2)
        out_wait(3)

    return k


def kernel(s1, s2, e1_weight, e2_weight):
    B, L = s1.shape
    D = e1_weight.shape[1]
    N = B * L
    s1f = s1.reshape(N).astype(jnp.int32)
    s2f = s2.reshape(N).astype(jnp.int32)
    k = _make_sc_kernel(N, D)
    out = k(s1f, s2f, e1_weight, e2_weight)
    return out.reshape(B, L, D)
